# baseline (speedup 1.0000x reference)
import functools

import jax
import jax.numpy as jnp
from jax.experimental import pallas as pl

DIM = 64
M_DIM = 16
HID = 2 * (2 * DIM + 2)


def _silu(x):
    return 0.5 * x * (1.0 + jnp.tanh(0.5 * x))


def _egnn_block_kernel(
    feats_i_ref, feats_all_ref,
    cm_i_ref, cmT_ref, cm_aug_ref,
    cv_i_ref, cvT_ref, cv_aug_ref,
    w1a_ref, w1b_ref, wdm_ref, wds_ref, b1_ref,
    w2_ref, b2_ref,
    hw1_ref, hb1_ref, hw2_ref, hb2_ref,
    nw1_ref, nb1_ref, nw2_ref, nb2_ref,
    node_out_ref, cm_out_ref, cv_out_ref,
    *, bi, n,
):
    f_i = feats_i_ref[0]
    f_all = feats_all_ref[0]

    a = jnp.dot(f_i, w1a_ref[...], preferred_element_type=jnp.float32)
    a = a + b1_ref[...]
    bmat = jnp.dot(f_all, w1b_ref[...], preferred_element_type=jnp.float32)

    cm_i = cm_i_ref[0]
    cv_i = cv_i_ref[0]
    cmT = cmT_ref[0]
    cvT = cvT_ref[0]
    dsum = jnp.zeros((bi, n), jnp.float32)
    vtr = jnp.zeros((bi, n), jnp.float32)
    q = jnp.zeros((bi, n), jnp.float32)
    for c in range(3):
        rel = cm_i[:, c:c + 1] - cmT[c:c + 1, :]
        rv = cv_i[:, c:c + 1] + cvT[c:c + 1, :]
        rel2 = rel * rel
        dsum = dsum + rel2
        vtr = vtr + rv
        q = q + rel2 * rv
    dm = dsum + vtr
    ds = 2.0 * vtr + 4.0 * q

    pre = (
        a[:, None, :]
        + bmat[None, :, :]
        + dm[:, :, None] * wdm_ref[...][None, :, :]
        + ds[:, :, None] * wds_ref[...][None, :, :]
    )
    h = _silu(pre).astype(jnp.bfloat16).reshape(bi * n, HID)

    m2 = jnp.dot(h, w2_ref[...], preferred_element_type=jnp.float32)
    m2 = _silu(m2 + b2_ref[...])

    hh = jnp.dot(m2.astype(jnp.bfloat16), hw1_ref[...],
                 preferred_element_type=jnp.float32)
    hh = _silu(hh + hb1_ref[...]).astype(jnp.bfloat16)
    wout = jnp.dot(hh, hw2_ref[...], preferred_element_type=jnp.float32)
    wout = wout + hb2_ref[...]
    wm = wout[:, 0:1].reshape(bi, n)
    wv = wout[:, 1:2].reshape(bi, n)
    wv2 = wv * wv

    rm = jnp.dot(wm, cm_aug_ref[0], preferred_element_type=jnp.float32)
    rv_ = jnp.dot(wv2, cv_aug_ref[0], preferred_element_type=jnp.float32)
    cm_out = cm_i + rm[:, 3:4] * cm_i - rm[:, 0:3]
    cv_out = cv_i + rv_[:, 3:4] * cv_i + rv_[:, 0:3]

    m_i = jnp.sum(m2.reshape(bi, n, M_DIM), axis=1)
    nin = jnp.concatenate([f_i, m_i], axis=1)
    nh = _silu(jnp.dot(nin, nw1_ref[...], preferred_element_type=jnp.float32)
               + nb1_ref[...])
    nout = jnp.dot(nh, nw2_ref[...], preferred_element_type=jnp.float32)
    nout = nout + nb2_ref[...] + f_i

    node_out_ref[0] = nout
    cm_out_ref[0] = cm_out
    cv_out_ref[0] = cv_out


@jax.jit
def kernel(feats, coors_mean, coors_var, params):
    b, n, d = feats.shape
    bi = 16

    w1 = params['edge_w1']
    w1a = jnp.transpose(w1[:, :DIM])
    w1b = jnp.transpose(w1[:, DIM:2 * DIM])
    wdm = w1[:, 2 * DIM].reshape(1, HID)
    wds = w1[:, 2 * DIM + 1].reshape(1, HID)
    b1 = params['edge_b1'].reshape(1, HID)
    w2 = jnp.transpose(params['edge_w2']).astype(jnp.bfloat16)
    b2 = params['edge_b2'].reshape(1, M_DIM)
    hw1 = jnp.concatenate(
        [jnp.transpose(params['cm_w1']), jnp.transpose(params['cv_w1'])],
        axis=1).astype(jnp.bfloat16)
    hb1 = jnp.concatenate(
        [params['cm_b1'], params['cv_b1']]).reshape(1, 8 * M_DIM)
    z64 = jnp.zeros((4 * M_DIM, 1), jnp.float32)
    hw2 = jnp.concatenate([
        jnp.concatenate([jnp.transpose(params['cm_w2']), z64], axis=0),
        jnp.concatenate([z64, jnp.transpose(params['cv_w2'])], axis=0),
    ], axis=1).astype(jnp.bfloat16)
    hb2 = jnp.concatenate(
        [params['cm_b2'], params['cv_b2']]).reshape(1, 2)
    nw1 = jnp.transpose(params['node_w1'])
    nb1 = params['node_b1'].reshape(1, 2 * DIM)
    nw2 = jnp.transpose(params['node_w2'])
    nb2 = params['node_b2'].reshape(1, DIM)

    cmT = jnp.transpose(coors_mean, (0, 2, 1))
    cvT = jnp.transpose(coors_var, (0, 2, 1))
    ones = jnp.ones((b, n, 1), jnp.float32)
    cm_aug = jnp.concatenate([coors_mean, ones], axis=2)
    cv_aug = jnp.concatenate([coors_var, ones], axis=2)

    grid = (b, n // bi)

    def im_block(ib, ii):
        return (ib, ii, 0)

    def im_batch(ib, ii):
        return (ib, 0, 0)

    def im_const(ib, ii):
        return (0, 0)

    full = lambda shape: pl.BlockSpec(shape, im_const)

    out_shapes = (
        jax.ShapeDtypeStruct((b, n, d), jnp.float32),
        jax.ShapeDtypeStruct((b, n, 3), jnp.float32),
        jax.ShapeDtypeStruct((b, n, 3), jnp.float32),
    )

    node_out, cm_out, cv_out = pl.pallas_call(
        functools.partial(_egnn_block_kernel, bi=bi, n=n),
        grid=grid,
        in_specs=[
            pl.BlockSpec((1, bi, d), im_block),
            pl.BlockSpec((1, n, d), im_batch),
            pl.BlockSpec((1, bi, 3), im_block),
            pl.BlockSpec((1, 3, n), im_batch),
            pl.BlockSpec((1, n, 4), im_batch),
            pl.BlockSpec((1, bi, 3), im_block),
            pl.BlockSpec((1, 3, n), im_batch),
            pl.BlockSpec((1, n, 4), im_batch),
            full((d, HID)), full((d, HID)),
            full((1, HID)), full((1, HID)), full((1, HID)),
            full((HID, M_DIM)), full((1, M_DIM)),
            full((M_DIM, 8 * M_DIM)), full((1, 8 * M_DIM)),
            full((8 * M_DIM, 2)), full((1, 2)),
            full((DIM + M_DIM, 2 * DIM)), full((1, 2 * DIM)),
            full((2 * DIM, DIM)), full((1, DIM)),
        ],
        out_specs=(
            pl.BlockSpec((1, bi, d), im_block),
            pl.BlockSpec((1, bi, 3), im_block),
            pl.BlockSpec((1, bi, 3), im_block),
        ),
        out_shape=out_shapes,
    )(
        feats, feats,
        coors_mean, cmT, cm_aug,
        coors_var, cvT, cv_aug,
        w1a, w1b, wdm, wds, b1,
        w2, b2,
        hw1, hb1, hw2, hb2,
        nw1, nb1, nw2, nb2,
    )
    return node_out, cm_out, cv_out


# speedup vs baseline: 1.9969x; 1.9969x over previous
import functools

import jax
import jax.numpy as jnp
from jax.experimental import pallas as pl

DIM = 64
M_DIM = 16
HID = 2 * (2 * DIM + 2)


def _silu(x):
    return 0.5 * x * (1.0 + jnp.tanh(0.5 * x))


def _egnn_block_kernel(
    fti_ref, fta_ref,
    cmi_ref, cmT_ref, cvi_ref, cvT_ref, aug_ref,
    w1a_ref, w1b_ref, wdm_ref, wds_ref, b1_ref,
    w2_ref, b2_ref,
    hw1_ref, hb1_ref, hw2_ref, hb2_ref,
    nw1_ref, nb1_ref, nw2_ref, nb2_ref,
    node_out_ref, cm_out_ref, cv_out_ref,
    *, bi, n,
):
    fti = fti_ref[0, 0]
    fta = fta_ref[0]

    at = jnp.dot(w1a_ref[...], fti, preferred_element_type=jnp.float32)
    at = (at + b1_ref[...]).astype(jnp.bfloat16)
    bt = jnp.dot(w1b_ref[...], fta,
                 preferred_element_type=jnp.float32).astype(jnp.bfloat16)

    cm_i = cmi_ref[0]
    cv_i = cvi_ref[0]
    cmT = cmT_ref[0]
    cvT = cvT_ref[0]
    dsum = jnp.zeros((bi, n), jnp.float32)
    vtr = jnp.zeros((bi, n), jnp.float32)
    q = jnp.zeros((bi, n), jnp.float32)
    for c in range(3):
        rel = cm_i[:, c:c + 1] - cmT[c:c + 1, :]
        rv = cv_i[:, c:c + 1] + cvT[c:c + 1, :]
        rel2 = rel * rel
        dsum = dsum + rel2
        vtr = vtr + rv
        q = q + rel2 * rv
    dm = (dsum + vtr).astype(jnp.bfloat16)
    ds = (2.0 * vtr + 4.0 * q).astype(jnp.bfloat16)

    aug = aug_ref[0]
    wdm = wdm_ref[...]
    wds = wds_ref[...]
    w2 = w2_ref[...]

    hs = []
    for i in range(bi):
        pre = (at[:, i:i + 1] + bt
               + dm[i:i + 1, :] * wdm
               + ds[i:i + 1, :] * wds)
        hs.append(_silu(pre))

    mts = [jnp.dot(w2, h, preferred_element_type=jnp.float32) for h in hs]
    mt_all = jnp.concatenate(mts, axis=0)
    mt_all = _silu(mt_all + b2_ref[...])

    hh = jnp.dot(hw1_ref[...], mt_all.astype(jnp.bfloat16),
                 preferred_element_type=jnp.float32)
    hh = _silu((hh + hb1_ref[...]).astype(jnp.bfloat16))
    wo = jnp.dot(hw2_ref[...], hh,
                 preferred_element_type=jnp.float32) + hb2_ref[...]

    cmat = jnp.concatenate([wo, wo * wo, mt_all], axis=0)
    s = jnp.dot(cmat, aug, preferred_element_type=jnp.float32)
    w8 = jnp.concatenate(
        [s[2 * i:2 * i + 1, :] for i in range(bi)], axis=0)
    v8 = jnp.concatenate(
        [s[2 * bi + 2 * i + 1:2 * bi + 2 * i + 2, :] for i in range(bi)],
        axis=0)
    moff = 4 * bi
    m_t = jnp.concatenate(
        [s[moff + M_DIM * i:moff + M_DIM * (i + 1), 3:4] for i in range(bi)],
        axis=1)

    cm_out = cm_i + w8[:, 3:4] * cm_i - w8[:, 0:3]
    cv_out = cv_i + v8[:, 7:8] * cv_i + v8[:, 4:7]

    nint = jnp.concatenate([fti, m_t], axis=0)
    nh = _silu(jnp.dot(nw1_ref[...], nint,
                       preferred_element_type=jnp.float32) + nb1_ref[...])
    nout = jnp.dot(nw2_ref[...], nh,
                   preferred_element_type=jnp.float32) + nb2_ref[...] + fti

    node_out_ref[0] = nout.T
    cm_out_ref[0] = cm_out
    cv_out_ref[0] = cv_out


@jax.jit
def kernel(feats, coors_mean, coors_var, params):
    b, n, d = feats.shape
    bi = 16

    w1 = params['edge_w1']
    w1a = w1[:, :DIM]
    w1b = w1[:, DIM:2 * DIM]
    wdm = w1[:, 2 * DIM:2 * DIM + 1].astype(jnp.bfloat16)
    wds = w1[:, 2 * DIM + 1:2 * DIM + 2].astype(jnp.bfloat16)
    b1 = params['edge_b1'].reshape(HID, 1)
    w2 = params['edge_w2'].astype(jnp.bfloat16)
    b2 = jnp.tile(params['edge_b2'].reshape(M_DIM, 1), (bi, 1))
    hw1_1 = jnp.concatenate(
        [params['cm_w1'], params['cv_w1']], axis=0)
    eye_bi = jnp.eye(bi, dtype=jnp.float32)
    hw1 = jnp.einsum('pq,ab->paqb', eye_bi, hw1_1).reshape(
        bi * 8 * M_DIM, bi * M_DIM).astype(jnp.bfloat16)
    hb1 = jnp.tile(jnp.concatenate(
        [params['cm_b1'], params['cv_b1']]).reshape(8 * M_DIM, 1), (bi, 1))
    z64 = jnp.zeros((1, 4 * M_DIM), jnp.float32)
    hw2_1 = jnp.concatenate([
        jnp.concatenate([params['cm_w2'], z64], axis=1),
        jnp.concatenate([z64, params['cv_w2']], axis=1),
    ], axis=0)
    hw2 = jnp.einsum('pq,ab->paqb', eye_bi, hw2_1).reshape(
        2 * bi, bi * 8 * M_DIM).astype(jnp.bfloat16)
    hb2 = jnp.tile(jnp.concatenate(
        [params['cm_b2'], params['cv_b2']]).reshape(2, 1), (bi, 1))
    nw1 = params['node_w1']
    nb1 = params['node_b1'].reshape(2 * DIM, 1)
    nw2 = params['node_w2']
    nb2 = params['node_b2'].reshape(DIM, 1)

    featsT = jnp.transpose(feats, (0, 2, 1))
    featsT_blk = jnp.transpose(
        featsT.reshape(b, d, n // bi, bi), (0, 2, 1, 3))
    cmT = jnp.transpose(coors_mean, (0, 2, 1))
    cvT = jnp.transpose(coors_var, (0, 2, 1))
    ones = jnp.ones((b, n, 1), jnp.float32)
    aug = jnp.concatenate([coors_mean, ones, coors_var, ones], axis=2)

    grid = (b, n // bi)

    def im_block(ib, ii):
        return (ib, ii, 0)

    def im_icol(ib, ii):
        return (ib, ii, 0, 0)

    def im_batch(ib, ii):
        return (ib, 0, 0)

    def im_const(ib, ii):
        return (0, 0)

    full = lambda shape: pl.BlockSpec(shape, im_const)

    out_shapes = (
        jax.ShapeDtypeStruct((b, n, d), jnp.float32),
        jax.ShapeDtypeStruct((b, n, 3), jnp.float32),
        jax.ShapeDtypeStruct((b, n, 3), jnp.float32),
    )

    node_out, cm_out, cv_out = pl.pallas_call(
        functools.partial(_egnn_block_kernel, bi=bi, n=n),
        grid=grid,
        in_specs=[
            pl.BlockSpec((1, 1, d, bi), im_icol),
            pl.BlockSpec((1, d, n), im_batch),
            pl.BlockSpec((1, bi, 3), im_block),
            pl.BlockSpec((1, 3, n), im_batch),
            pl.BlockSpec((1, bi, 3), im_block),
            pl.BlockSpec((1, 3, n), im_batch),
            pl.BlockSpec((1, n, 8), im_batch),
            full((HID, DIM)), full((HID, DIM)),
            full((HID, 1)), full((HID, 1)), full((HID, 1)),
            full((M_DIM, HID)), full((bi * M_DIM, 1)),
            full((bi * 8 * M_DIM, bi * M_DIM)),
            full((bi * 8 * M_DIM, 1)),
            full((2 * bi, bi * 8 * M_DIM)), full((2 * bi, 1)),
            full((2 * DIM, DIM + M_DIM)), full((2 * DIM, 1)),
            full((DIM, 2 * DIM)), full((DIM, 1)),
        ],
        out_specs=(
            pl.BlockSpec((1, bi, d), im_block),
            pl.BlockSpec((1, bi, 3), im_block),
            pl.BlockSpec((1, bi, 3), im_block),
        ),
        out_shape=out_shapes,
    )(
        featsT_blk, featsT,
        coors_mean, cmT, coors_var, cvT, aug,
        w1a, w1b, wdm, wds, b1,
        w2, b2,
        hw1, hb1, hw2, hb2,
        nw1, nb1, nw2, nb2,
    )
    return node_out, cm_out, cv_out
